# baseline (speedup 1.0000x reference)
import functools

import jax
import jax.numpy as jnp
from jax import lax
from jax.experimental import pallas as pl
from jax.experimental.pallas import tpu as pltpu
from jax.experimental.pallas import tpu_sc as plsc

_H = 768
_E = 64
_K = 8
_F = 768
_SF = 2048
_T = 256
_TM = 64
_NBUF = 3

_NC = 2
_NS = 16
_NW = _NC * _NS
_TPW = _T // _NW
_EPW = _E // _NW
_NB4 = _E // 16


def _sig(v):
    return 1.0 / (1.0 + jnp.exp(-v))


def _probs_shared_body(x_ref, gw_ref, swg_ref, swu_ref, swd_ref, sgw_ref,
                       probs_ref, sh_ref):
    x = x_ref[...]
    logits = lax.dot_general(x, gw_ref[...], (((1,), (1,)), ((), ())),
                             preferred_element_type=jnp.float32)
    m = jnp.max(logits, axis=1, keepdims=True)
    p = jnp.exp(logits - m)
    probs_ref[...] = p / jnp.sum(p, axis=1, keepdims=True)
    sg = jnp.dot(x, swg_ref[...], preferred_element_type=jnp.float32)
    su = jnp.dot(x, swu_ref[...], preferred_element_type=jnp.float32)
    sh = (sg * _sig(sg)) * su
    so = jnp.dot(sh, swd_ref[...], preferred_element_type=jnp.float32)
    gate = jnp.sum(x * jnp.broadcast_to(sgw_ref[...], (_T, _H)),
                   axis=1, keepdims=True)
    sh_ref[...] = _sig(gate) * so


def _bmax(v):
    return plsc.cummax(lax.rev(plsc.cummax(v), (0,)))


def _bmin(v):
    return -_bmax(-v)


def _sc_topk_body(probs_hbm, mask_hbm, comb_hbm, pvm, mvm, cvm):
    w = lax.axis_index("s") * _NC + lax.axis_index("c")
    pltpu.sync_copy(probs_hbm.at[pl.ds(w * _TPW, _TPW)], pvm)
    iot = lax.iota(jnp.int32, 16).astype(jnp.float32)
    for t in range(_TPW):
        banks = [pvm[t, pl.ds(b * 16, 16)] for b in range(_NB4)]
        orig = list(banks)
        macc = [jnp.zeros((16,), jnp.float32) for _ in range(_NB4)]
        for _ in range(_K):
            mx = banks[0]
            for b in range(1, _NB4):
                mx = jnp.maximum(mx, banks[b])
            mval = _bmax(mx)
            jm = jnp.full((16,), float(_E), jnp.float32)
            for b in range(_NB4):
                cand = jnp.where(banks[b] == mval,
                                 iot + jnp.float32(b * 16),
                                 jnp.float32(float(_E)))
                jm = jnp.minimum(jm, cand)
            jm = _bmin(jm)
            for b in range(_NB4):
                oh = (iot + jnp.float32(b * 16)) == jm
                macc[b] = macc[b] + oh.astype(jnp.float32)
                banks[b] = jnp.where(oh, jnp.float32(-1.0), banks[b])
        for b in range(_NB4):
            mvm[t, pl.ds(b * 16, 16)] = macc[b]
            cvm[t, pl.ds(b * 16, 16)] = macc[b] * orig[b]
    pltpu.sync_copy(mvm, mask_hbm.at[pl.ds(w * _TPW, _TPW)])
    pltpu.sync_copy(cvm, comb_hbm.at[pl.ds(w * _TPW, _TPW)])


def _sc_ranks_body(mask_hbm, comb_hbm, rm_hbm, cb_hbm, meta_hbm,
                   mall, call, rmv, cbv, mtv):
    w = lax.axis_index("s") * _NC + lax.axis_index("c")
    pltpu.sync_copy(mask_hbm, mall)
    pltpu.sync_copy(comb_hbm, call)
    iot = lax.iota(jnp.int32, 16)
    for i in range(_EPW):
        e = w * _EPW + i
        eidx = jnp.full((16,), e, jnp.int32)
        carry = jnp.zeros((16,), jnp.float32)
        for c in range(_T // 16):
            tidx = iot + jnp.int32(c * 16)
            mk = plsc.load_gather(mall, [tidx, eidx])
            cbk = plsc.load_gather(call, [tidx, eidx])
            r = plsc.cumsum(mk) + carry
            carry = _bmax(r)
            rmv[i, pl.ds(c * 16, 16)] = jnp.where(mk > 0, r, 0.0)
            cbv[i, pl.ds(c * 16, 16)] = cbk
        ci = carry.astype(jnp.int32)
        nb = (ci + (_TM - 1)) // _TM
        mtv[i, :] = jnp.where(iot == 0, nb, 0)
    pltpu.sync_copy(rmv, rm_hbm.at[pl.ds(w * _EPW, _EPW)])
    pltpu.sync_copy(cbv, cb_hbm.at[pl.ds(w * _EPW, _EPW)])
    pltpu.sync_copy(mtv, meta_hbm.at[pl.ds(w * _EPW, _EPW)])


def _moe_body(meta_ref, x_ref, sh_ref, rm_ref, cb_ref, wg_ref, wu_ref, wd_ref,
              out_ref, wgb, wub, wdb, sems):
    e = pl.program_id(0)

    def issue(idx):
        slot = lax.rem(idx, _NBUF)
        pltpu.make_async_copy(wg_ref.at[idx], wgb.at[slot],
                              sems.at[slot, 0]).start()
        pltpu.make_async_copy(wu_ref.at[idx], wub.at[slot],
                              sems.at[slot, 1]).start()
        pltpu.make_async_copy(wd_ref.at[idx], wdb.at[slot],
                              sems.at[slot, 2]).start()

    @pl.when(e == 0)
    def _():
        out_ref[...] = sh_ref[...]
        issue(0)
        issue(1)

    @pl.when(e + 2 < _E)
    def _():
        issue(e + 2)

    slot = lax.rem(e, _NBUF)
    pltpu.make_async_copy(wg_ref.at[e], wgb.at[slot], sems.at[slot, 0]).wait()
    pltpu.make_async_copy(wu_ref.at[e], wub.at[slot], sems.at[slot, 1]).wait()
    pltpu.make_async_copy(wd_ref.at[e], wdb.at[slot], sems.at[slot, 2]).wait()

    nblk = meta_ref[e, 0]

    for j in range(_T // _TM):
        @pl.when(j < nblk)
        def _(j=j):
            rm = jnp.broadcast_to(rm_ref[pl.ds(e, 1), :], (_TM, _T))
            cb = jnp.broadcast_to(cb_ref[pl.ds(e, 1), :], (_TM, _T))
            pos = float(j * _TM + 1) + \
                lax.broadcasted_iota(jnp.int32, (_TM, _T), 0).astype(
                    jnp.float32)
            P = (rm == pos).astype(jnp.float32)
            X = jnp.dot(P, x_ref[...], preferred_element_type=jnp.float32)
            wg = wgb[pl.ds(slot, 1)][0].astype(jnp.bfloat16)
            wu = wub[pl.ds(slot, 1)][0].astype(jnp.bfloat16)
            wd = wdb[pl.ds(slot, 1)][0].astype(jnp.bfloat16)
            Xb = X.astype(jnp.bfloat16)
            g = jnp.dot(Xb, wg, preferred_element_type=jnp.float32)
            u = jnp.dot(Xb, wu, preferred_element_type=jnp.float32)
            h = (g * _sig(g)) * u
            o = jnp.dot(h.astype(jnp.bfloat16), wd,
                        preferred_element_type=jnp.float32)
            contrib = lax.dot_general(P * cb, o,
                                      (((0,), (0,)), ((), ())),
                                      preferred_element_type=jnp.float32)
            out_ref[...] += contrib


def kernel(x, gate_w, w_gate, w_up, w_down, sw_gate, sw_up, sw_down,
           shared_gate_w):
    probs, shared = pl.pallas_call(
        _probs_shared_body,
        out_shape=(
            jax.ShapeDtypeStruct((_T, _E), jnp.float32),
            jax.ShapeDtypeStruct((_T, _H), jnp.float32),
        ),
    )(x, gate_w, sw_gate, sw_up, sw_down, shared_gate_w)

    mesh = plsc.VectorSubcoreMesh(core_axis_name="c", subcore_axis_name="s")

    topk = functools.partial(
        pl.kernel,
        out_type=(
            jax.ShapeDtypeStruct((_T, _E), jnp.float32),
            jax.ShapeDtypeStruct((_T, _E), jnp.float32),
        ),
        mesh=mesh,
        compiler_params=pltpu.CompilerParams(needs_layout_passes=False),
        scratch_types=[
            pltpu.VMEM((_TPW, _E), jnp.float32),
            pltpu.VMEM((_TPW, _E), jnp.float32),
            pltpu.VMEM((_TPW, _E), jnp.float32),
        ],
    )(_sc_topk_body)
    maskf, comb = topk(probs)

    ranks = functools.partial(
        pl.kernel,
        out_type=(
            jax.ShapeDtypeStruct((_E, _T), jnp.float32),
            jax.ShapeDtypeStruct((_E, _T), jnp.float32),
            jax.ShapeDtypeStruct((_E, 16), jnp.int32),
        ),
        mesh=mesh,
        compiler_params=pltpu.CompilerParams(needs_layout_passes=False),
        scratch_types=[
            pltpu.VMEM((_T, _E), jnp.float32),
            pltpu.VMEM((_T, _E), jnp.float32),
            pltpu.VMEM((_EPW, _T), jnp.float32),
            pltpu.VMEM((_EPW, _T), jnp.float32),
            pltpu.VMEM((_EPW, 16), jnp.int32),
        ],
    )(_sc_ranks_body)
    rm, cb, meta = ranks(maskf, comb)

    grid_spec = pltpu.PrefetchScalarGridSpec(
        num_scalar_prefetch=1,
        grid=(_E,),
        in_specs=[
            pl.BlockSpec((_T, _H), lambda e, m: (0, 0)),
            pl.BlockSpec((_T, _H), lambda e, m: (0, 0)),
            pl.BlockSpec((_E, _T), lambda e, m: (0, 0)),
            pl.BlockSpec((_E, _T), lambda e, m: (0, 0)),
            pl.BlockSpec(memory_space=pl.ANY),
            pl.BlockSpec(memory_space=pl.ANY),
            pl.BlockSpec(memory_space=pl.ANY),
        ],
        out_specs=pl.BlockSpec((_T, _H), lambda e, m: (0, 0)),
        scratch_shapes=[
            pltpu.VMEM((_NBUF, _H, _F), jnp.float32),
            pltpu.VMEM((_NBUF, _H, _F), jnp.float32),
            pltpu.VMEM((_NBUF, _F, _H), jnp.float32),
            pltpu.SemaphoreType.DMA((_NBUF, 3)),
        ],
    )
    out = pl.pallas_call(
        _moe_body,
        grid_spec=grid_spec,
        out_shape=jax.ShapeDtypeStruct((_T, _H), jnp.float32),
        compiler_params=pltpu.CompilerParams(
            dimension_semantics=("arbitrary",)),
    )(meta, x, shared, rm, cb, w_gate, w_up, w_down)
    return out


# speedup vs baseline: 1.0689x; 1.0689x over previous
import functools

import jax
import jax.numpy as jnp
from jax import lax
from jax.experimental import pallas as pl
from jax.experimental.pallas import tpu as pltpu
from jax.experimental.pallas import tpu_sc as plsc

_H = 768
_E = 64
_K = 8
_F = 768
_SF = 2048
_T = 256
_TM = 64
_NBUF = 3

_NC = 2
_NS = 16
_NW = _NC * _NS
_TPW = _T // _NW
_EPW = _E // _NW
_NB4 = _E // 16


def _sig(v):
    return 1.0 / (1.0 + jnp.exp(-v))


def _probs_body(x_ref, gw_ref, probs_ref):
    x = x_ref[...]
    logits = lax.dot_general(x, gw_ref[...], (((1,), (1,)), ((), ())),
                             preferred_element_type=jnp.float32)
    m = jnp.max(logits, axis=1, keepdims=True)
    p = jnp.exp(logits - m)
    probs_ref[...] = p / jnp.sum(p, axis=1, keepdims=True)


def _shared_body(x_ref, swg_ref, swu_ref, swd_ref, sgw_ref, sh_ref):
    x = x_ref[...]
    sg = jnp.dot(x, swg_ref[...], preferred_element_type=jnp.float32)
    su = jnp.dot(x, swu_ref[...], preferred_element_type=jnp.float32)
    sh = (sg * _sig(sg)) * su
    so = jnp.dot(sh, swd_ref[...], preferred_element_type=jnp.float32)
    gate = jnp.sum(x * jnp.broadcast_to(sgw_ref[...], (_T, _H)),
                   axis=1, keepdims=True)
    sh_ref[...] = _sig(gate) * so


def _bmax(v):
    return plsc.cummax(lax.rev(plsc.cummax(v), (0,)))


def _bmin(v):
    return -_bmax(-v)


def _sc_topk_body(probs_hbm, comb_hbm, pvm, cvm):
    w = lax.axis_index("s") * _NC + lax.axis_index("c")
    pltpu.sync_copy(probs_hbm.at[pl.ds(w * _TPW, _TPW)], pvm)
    iot = lax.iota(jnp.int32, 16).astype(jnp.float32)
    for t in range(_TPW):
        banks = [pvm[t, pl.ds(b * 16, 16)] for b in range(_NB4)]
        orig = list(banks)
        macc = [jnp.zeros((16,), jnp.float32) for _ in range(_NB4)]
        for _ in range(_K):
            mx = banks[0]
            for b in range(1, _NB4):
                mx = jnp.maximum(mx, banks[b])
            mval = _bmax(mx)
            jm = jnp.full((16,), float(_E), jnp.float32)
            for b in range(_NB4):
                cand = jnp.where(banks[b] == mval,
                                 iot + jnp.float32(b * 16),
                                 jnp.float32(float(_E)))
                jm = jnp.minimum(jm, cand)
            jm = _bmin(jm)
            for b in range(_NB4):
                oh = (iot + jnp.float32(b * 16)) == jm
                macc[b] = macc[b] + oh.astype(jnp.float32)
                banks[b] = jnp.where(oh, jnp.float32(-1.0), banks[b])
        for b in range(_NB4):
            cvm[t, pl.ds(b * 16, 16)] = macc[b] * orig[b]
    pltpu.sync_copy(cvm, comb_hbm.at[pl.ds(w * _TPW, _TPW)])


def _sc_ranks_body(comb_hbm, rm_hbm, cb_hbm, meta_hbm,
                   call, rmv, cbv, mtv):
    w = lax.axis_index("s") * _NC + lax.axis_index("c")
    pltpu.sync_copy(comb_hbm, call)
    iot = lax.iota(jnp.int32, 16)
    for i in range(_EPW):
        e = w * _EPW + i
        eidx = jnp.full((16,), e, jnp.int32)
        carry = jnp.zeros((16,), jnp.float32)
        for c in range(_T // 16):
            tidx = iot + jnp.int32(c * 16)
            cbk = plsc.load_gather(call, [tidx, eidx])
            mk = (cbk > 0).astype(jnp.float32)
            r = plsc.cumsum(mk) + carry
            carry = _bmax(r)
            rmv[i, pl.ds(c * 16, 16)] = jnp.where(mk > 0, r, 0.0)
            cbv[i, pl.ds(c * 16, 16)] = cbk
        ci = carry.astype(jnp.int32)
        nb = (ci + (_TM - 1)) // _TM
        mtv[i, :] = jnp.where(iot == 0, nb, 0)
    pltpu.sync_copy(rmv, rm_hbm.at[pl.ds(w * _EPW, _EPW)])
    pltpu.sync_copy(cbv, cb_hbm.at[pl.ds(w * _EPW, _EPW)])
    pltpu.sync_copy(mtv, meta_hbm.at[pl.ds(w * _EPW, _EPW)])


def _moe_body(meta_ref, x_ref, sh_ref, rm_ref, cb_ref, wg_ref, wu_ref, wd_ref,
              out_ref, wgb, wub, wdb, sems):
    e = pl.program_id(0)

    def issue(idx):
        slot = lax.rem(idx, _NBUF)
        pltpu.make_async_copy(wg_ref.at[idx], wgb.at[slot],
                              sems.at[slot, 0]).start()
        pltpu.make_async_copy(wu_ref.at[idx], wub.at[slot],
                              sems.at[slot, 1]).start()
        pltpu.make_async_copy(wd_ref.at[idx], wdb.at[slot],
                              sems.at[slot, 2]).start()

    @pl.when(e == 0)
    def _():
        out_ref[...] = sh_ref[...]
        issue(0)
        issue(1)

    @pl.when(e + 2 < _E)
    def _():
        issue(e + 2)

    slot = lax.rem(e, _NBUF)
    pltpu.make_async_copy(wg_ref.at[e], wgb.at[slot], sems.at[slot, 0]).wait()
    pltpu.make_async_copy(wu_ref.at[e], wub.at[slot], sems.at[slot, 1]).wait()
    pltpu.make_async_copy(wd_ref.at[e], wdb.at[slot], sems.at[slot, 2]).wait()

    nblk = meta_ref[e, 0]

    for j in range(_T // _TM):
        @pl.when(j < nblk)
        def _(j=j):
            rm = jnp.broadcast_to(rm_ref[pl.ds(e, 1), :], (_TM, _T))
            cb = jnp.broadcast_to(cb_ref[pl.ds(e, 1), :], (_TM, _T))
            pos = float(j * _TM + 1) + \
                lax.broadcasted_iota(jnp.int32, (_TM, _T), 0).astype(
                    jnp.float32)
            P = (rm == pos).astype(jnp.float32)
            X = jnp.dot(P, x_ref[...], preferred_element_type=jnp.float32)
            wg = wgb[pl.ds(slot, 1)][0].astype(jnp.bfloat16)
            wu = wub[pl.ds(slot, 1)][0].astype(jnp.bfloat16)
            wd = wdb[pl.ds(slot, 1)][0].astype(jnp.bfloat16)
            Xb = X.astype(jnp.bfloat16)
            g = jnp.dot(Xb, wg, preferred_element_type=jnp.float32)
            u = jnp.dot(Xb, wu, preferred_element_type=jnp.float32)
            h = (g * _sig(g)) * u
            o = jnp.dot(h.astype(jnp.bfloat16), wd,
                        preferred_element_type=jnp.float32)
            contrib = lax.dot_general(P * cb, o,
                                      (((0,), (0,)), ((), ())),
                                      preferred_element_type=jnp.float32)
            out_ref[...] += contrib


def kernel(x, gate_w, w_gate, w_up, w_down, sw_gate, sw_up, sw_down,
           shared_gate_w):
    probs = pl.pallas_call(
        _probs_body,
        out_shape=jax.ShapeDtypeStruct((_T, _E), jnp.float32),
    )(x, gate_w)

    mesh = plsc.VectorSubcoreMesh(core_axis_name="c", subcore_axis_name="s")

    topk = functools.partial(
        pl.kernel,
        out_type=jax.ShapeDtypeStruct((_T, _E), jnp.float32),
        mesh=mesh,
        compiler_params=pltpu.CompilerParams(needs_layout_passes=False),
        scratch_types=[
            pltpu.VMEM((_TPW, _E), jnp.float32),
            pltpu.VMEM((_TPW, _E), jnp.float32),
        ],
    )(_sc_topk_body)
    comb = topk(probs)

    ranks = functools.partial(
        pl.kernel,
        out_type=(
            jax.ShapeDtypeStruct((_E, _T), jnp.float32),
            jax.ShapeDtypeStruct((_E, _T), jnp.float32),
            jax.ShapeDtypeStruct((_E, 16), jnp.int32),
        ),
        mesh=mesh,
        compiler_params=pltpu.CompilerParams(needs_layout_passes=False),
        scratch_types=[
            pltpu.VMEM((_T, _E), jnp.float32),
            pltpu.VMEM((_EPW, _T), jnp.float32),
            pltpu.VMEM((_EPW, _T), jnp.float32),
            pltpu.VMEM((_EPW, 16), jnp.int32),
        ],
    )(_sc_ranks_body)
    rm, cb, meta = ranks(comb)

    shared = pl.pallas_call(
        _shared_body,
        out_shape=jax.ShapeDtypeStruct((_T, _H), jnp.float32),
    )(x, sw_gate, sw_up, sw_down, shared_gate_w)

    grid_spec = pltpu.PrefetchScalarGridSpec(
        num_scalar_prefetch=1,
        grid=(_E,),
        in_specs=[
            pl.BlockSpec((_T, _H), lambda e, m: (0, 0)),
            pl.BlockSpec((_T, _H), lambda e, m: (0, 0)),
            pl.BlockSpec((_E, _T), lambda e, m: (0, 0)),
            pl.BlockSpec((_E, _T), lambda e, m: (0, 0)),
            pl.BlockSpec(memory_space=pl.ANY),
            pl.BlockSpec(memory_space=pl.ANY),
            pl.BlockSpec(memory_space=pl.ANY),
        ],
        out_specs=pl.BlockSpec((_T, _H), lambda e, m: (0, 0)),
        scratch_shapes=[
            pltpu.VMEM((_NBUF, _H, _F), jnp.float32),
            pltpu.VMEM((_NBUF, _H, _F), jnp.float32),
            pltpu.VMEM((_NBUF, _F, _H), jnp.float32),
            pltpu.SemaphoreType.DMA((_NBUF, 3)),
        ],
    )
    out = pl.pallas_call(
        _moe_body,
        grid_spec=grid_spec,
        out_shape=jax.ShapeDtypeStruct((_T, _H), jnp.float32),
        compiler_params=pltpu.CompilerParams(
            dimension_semantics=("arbitrary",)),
    )(meta, x, shared, rm, cb, w_gate, w_up, w_down)
    return out


# speedup vs baseline: 1.1252x; 1.0527x over previous
import functools

import jax
import jax.numpy as jnp
from jax import lax
from jax.experimental import pallas as pl
from jax.experimental.pallas import tpu as pltpu
from jax.experimental.pallas import tpu_sc as plsc

_H = 768
_E = 64
_K = 8
_F = 768
_SF = 2048
_T = 256
_TM = 64
_NBUF = 3

_NC = 2
_NS = 16
_NW = _NC * _NS
_TPW = _T // _NW
_EPW = _E // _NW
_NB4 = _E // 16


def _sig(v):
    return 1.0 / (1.0 + jnp.exp(-v))


def _probs_body(x_ref, gw_ref, probs_ref):
    x = x_ref[...]
    logits = lax.dot_general(x, gw_ref[...], (((1,), (1,)), ((), ())),
                             preferred_element_type=jnp.float32)
    m = jnp.max(logits, axis=1, keepdims=True)
    p = jnp.exp(logits - m)
    probs_ref[...] = p / jnp.sum(p, axis=1, keepdims=True)


def _shared_body(x_ref, swg_ref, swu_ref, swd_ref, sgw_ref, sh_ref):
    x = x_ref[...]
    sg = jnp.dot(x, swg_ref[...], preferred_element_type=jnp.float32)
    su = jnp.dot(x, swu_ref[...], preferred_element_type=jnp.float32)
    sh = (sg * _sig(sg)) * su
    so = jnp.dot(sh, swd_ref[...], preferred_element_type=jnp.float32)
    gate = jnp.sum(x * jnp.broadcast_to(sgw_ref[...], (_T, _H)),
                   axis=1, keepdims=True)
    sh_ref[...] = _sig(gate) * so


def _bmax(v):
    return plsc.cummax(lax.rev(plsc.cummax(v), (0,)))


def _bmin(v):
    return -_bmax(-v)


def _sc_topk_body(probs_hbm, comb_hbm, pvm, cvm):
    w = lax.axis_index("s") * _NC + lax.axis_index("c")
    pltpu.sync_copy(probs_hbm.at[pl.ds(w * _TPW, _TPW)], pvm)
    iot = lax.iota(jnp.int32, 16).astype(jnp.float32)
    for t in range(_TPW):
        banks = [pvm[t, pl.ds(b * 16, 16)] for b in range(_NB4)]
        orig = list(banks)
        macc = [jnp.zeros((16,), jnp.float32) for _ in range(_NB4)]
        for _ in range(_K):
            mx = banks[0]
            for b in range(1, _NB4):
                mx = jnp.maximum(mx, banks[b])
            mval = _bmax(mx)
            jm = jnp.full((16,), float(_E), jnp.float32)
            for b in range(_NB4):
                cand = jnp.where(banks[b] == mval,
                                 iot + jnp.float32(b * 16),
                                 jnp.float32(float(_E)))
                jm = jnp.minimum(jm, cand)
            jm = _bmin(jm)
            for b in range(_NB4):
                oh = (iot + jnp.float32(b * 16)) == jm
                macc[b] = macc[b] + oh.astype(jnp.float32)
                banks[b] = jnp.where(oh, jnp.float32(-1.0), banks[b])
        for b in range(_NB4):
            cvm[t, pl.ds(b * 16, 16)] = macc[b] * orig[b]
    pltpu.sync_copy(cvm, comb_hbm.at[pl.ds(w * _TPW, _TPW)])


def _prep_body(cb_ref, meta_ref, rmT_ref, cbT_ref):
    cb = cb_ref[...]
    ia = lax.broadcasted_iota(jnp.int32, (_T, _T), 0)
    ib = lax.broadcasted_iota(jnp.int32, (_T, _T), 1)
    ident = (ia == ib).astype(jnp.float32)
    tri = (ia <= ib).astype(jnp.float32)
    cbT = lax.dot_general(cb, ident, (((0,), (0,)), ((), ())),
                          preferred_element_type=jnp.float32)
    maskT = (cbT > 0).astype(jnp.float32)
    rT = jnp.dot(maskT, tri, preferred_element_type=jnp.float32)
    rmT_ref[...] = jnp.where(maskT > 0, rT, 0.0)
    cbT_ref[...] = cbT
    counts = jnp.sum(maskT, axis=1, keepdims=True)
    nb = jnp.floor((counts + (_TM - 1)) / _TM)
    lane = lax.broadcasted_iota(jnp.int32, (_E, 16), 1)
    meta_ref[...] = jnp.where(lane == 0, jnp.broadcast_to(nb, (_E, 16)),
                              0.0).astype(jnp.int32)


def _moe_body(meta_ref, x_ref, sh_ref, rm_ref, cb_ref, wg_ref, wu_ref, wd_ref,
              out_ref, wgb, wub, wdb, sems):
    e = pl.program_id(0)

    def issue(idx):
        slot = lax.rem(idx, _NBUF)
        pltpu.make_async_copy(wg_ref.at[idx], wgb.at[slot],
                              sems.at[slot, 0]).start()
        pltpu.make_async_copy(wu_ref.at[idx], wub.at[slot],
                              sems.at[slot, 1]).start()
        pltpu.make_async_copy(wd_ref.at[idx], wdb.at[slot],
                              sems.at[slot, 2]).start()

    @pl.when(e == 0)
    def _():
        out_ref[...] = sh_ref[...]
        issue(0)
        issue(1)

    @pl.when(e + 2 < _E)
    def _():
        issue(e + 2)

    slot = lax.rem(e, _NBUF)
    pltpu.make_async_copy(wg_ref.at[e], wgb.at[slot], sems.at[slot, 0]).wait()
    pltpu.make_async_copy(wu_ref.at[e], wub.at[slot], sems.at[slot, 1]).wait()
    pltpu.make_async_copy(wd_ref.at[e], wdb.at[slot], sems.at[slot, 2]).wait()

    nblk = meta_ref[e, 0]

    for j in range(_T // _TM):
        @pl.when(j < nblk)
        def _(j=j):
            rm = jnp.broadcast_to(rm_ref[pl.ds(e, 1), :], (_TM, _T))
            cb = jnp.broadcast_to(cb_ref[pl.ds(e, 1), :], (_TM, _T))
            pos = float(j * _TM + 1) + \
                lax.broadcasted_iota(jnp.int32, (_TM, _T), 0).astype(
                    jnp.float32)
            P = (rm == pos).astype(jnp.float32)
            X = jnp.dot(P, x_ref[...], preferred_element_type=jnp.float32)
            wg = wgb[pl.ds(slot, 1)][0].astype(jnp.bfloat16)
            wu = wub[pl.ds(slot, 1)][0].astype(jnp.bfloat16)
            wd = wdb[pl.ds(slot, 1)][0].astype(jnp.bfloat16)
            Xb = X.astype(jnp.bfloat16)
            g = jnp.dot(Xb, wg, preferred_element_type=jnp.float32)
            u = jnp.dot(Xb, wu, preferred_element_type=jnp.float32)
            h = (g * _sig(g)) * u
            o = jnp.dot(h.astype(jnp.bfloat16), wd,
                        preferred_element_type=jnp.float32)
            contrib = lax.dot_general(P * cb, o,
                                      (((0,), (0,)), ((), ())),
                                      preferred_element_type=jnp.float32)
            out_ref[...] += contrib


def kernel(x, gate_w, w_gate, w_up, w_down, sw_gate, sw_up, sw_down,
           shared_gate_w):
    probs = pl.pallas_call(
        _probs_body,
        out_shape=jax.ShapeDtypeStruct((_T, _E), jnp.float32),
    )(x, gate_w)

    mesh = plsc.VectorSubcoreMesh(core_axis_name="c", subcore_axis_name="s")

    topk = functools.partial(
        pl.kernel,
        out_type=jax.ShapeDtypeStruct((_T, _E), jnp.float32),
        mesh=mesh,
        compiler_params=pltpu.CompilerParams(needs_layout_passes=False),
        scratch_types=[
            pltpu.VMEM((_TPW, _E), jnp.float32),
            pltpu.VMEM((_TPW, _E), jnp.float32),
        ],
    )(_sc_topk_body)
    comb = topk(probs)

    shared = pl.pallas_call(
        _shared_body,
        out_shape=jax.ShapeDtypeStruct((_T, _H), jnp.float32),
    )(x, sw_gate, sw_up, sw_down, shared_gate_w)

    meta, rm, cb = pl.pallas_call(
        _prep_body,
        out_shape=(
            jax.ShapeDtypeStruct((_E, 16), jnp.int32),
            jax.ShapeDtypeStruct((_E, _T), jnp.float32),
            jax.ShapeDtypeStruct((_E, _T), jnp.float32),
        ),
    )(comb)

    grid_spec = pltpu.PrefetchScalarGridSpec(
        num_scalar_prefetch=1,
        grid=(_E,),
        in_specs=[
            pl.BlockSpec((_T, _H), lambda e, m: (0, 0)),
            pl.BlockSpec((_T, _H), lambda e, m: (0, 0)),
            pl.BlockSpec((_E, _T), lambda e, m: (0, 0)),
            pl.BlockSpec((_E, _T), lambda e, m: (0, 0)),
            pl.BlockSpec(memory_space=pl.ANY),
            pl.BlockSpec(memory_space=pl.ANY),
            pl.BlockSpec(memory_space=pl.ANY),
        ],
        out_specs=pl.BlockSpec((_T, _H), lambda e, m: (0, 0)),
        scratch_shapes=[
            pltpu.VMEM((_NBUF, _H, _F), jnp.float32),
            pltpu.VMEM((_NBUF, _H, _F), jnp.float32),
            pltpu.VMEM((_NBUF, _F, _H), jnp.float32),
            pltpu.SemaphoreType.DMA((_NBUF, 3)),
        ],
    )
    out = pl.pallas_call(
        _moe_body,
        grid_spec=grid_spec,
        out_shape=jax.ShapeDtypeStruct((_T, _H), jnp.float32),
        compiler_params=pltpu.CompilerParams(
            dimension_semantics=("arbitrary",)),
    )(meta, x, shared, rm, cb, w_gate, w_up, w_down)
    return out
